# baseline (speedup 1.0000x reference)
import jax
import jax.numpy as jnp
from jax.experimental import pallas as pl
from jax.experimental.pallas import tpu as pltpu

_B, _C_IN, _N = 8, 256, 196
_D, _K = 64, 1024
_BN = _B * _N


def _split3(x):
    h = x.astype(jnp.bfloat16)
    r = x - h.astype(jnp.float32)
    m = r.astype(jnp.bfloat16)
    l = (r - m.astype(jnp.float32)).astype(jnp.bfloat16)
    return h, m, l


def _vq_body(z_hbm, w_ref, emb_ref, out_ref, zbuf, sem):
    copies = [pltpu.make_async_copy(z_hbm.at[b], zbuf.at[b], sem)
              for b in range(_B)]
    for c in copies:
        c.start()

    emb = emb_ref[...]
    eh = emb.astype(jnp.bfloat16)
    el = (emb - eh.astype(jnp.float32)).astype(jnp.bfloat16)
    e_sq = jnp.sum(emb * emb, axis=1, keepdims=True)
    qh, qm, ql = _split3(e_sq)
    lhs = jnp.concatenate([-2.0 * eh, -2.0 * eh, -2.0 * el, qh, qm, ql],
                          axis=1)
    gather_lhs = jnp.concatenate([eh.T, el.T], axis=0)

    wb = w_ref[...].astype(jnp.bfloat16)
    zes = []
    for b in range(_B):
        copies[b].wait()
        zes.append(jnp.dot(wb, zbuf[b].astype(jnp.bfloat16),
                           preferred_element_type=jnp.float32))
    ze = jnp.concatenate(zes, axis=1)

    zh = ze.astype(jnp.bfloat16)
    zl = (ze - zh.astype(jnp.float32)).astype(jnp.bfloat16)
    ones = jnp.ones((1, _BN), dtype=jnp.bfloat16)
    rhs = jnp.concatenate([zh, zl, zh, ones, ones, ones], axis=0)
    s = jnp.dot(lhs, rhs, preferred_element_type=jnp.float32)

    m = jnp.min(s, axis=0, keepdims=True)
    kio = jax.lax.broadcasted_iota(jnp.int32, (_K, _BN), 0)
    idx = jnp.min(jnp.where(s <= m, kio, _K), axis=0)
    onehot = (kio == idx[None, :]).astype(jnp.bfloat16)
    zq2 = jnp.dot(gather_lhs, onehot,
                  preferred_element_type=jnp.float32)
    zq = zq2[:_D] + zq2[_D:]
    for b in range(_B):
        out_ref[b] = zq[:, b * _N:(b + 1) * _N]


def kernel(z, W, emb):
    return pl.pallas_call(
        _vq_body,
        in_specs=[
            pl.BlockSpec(memory_space=pl.ANY),
            pl.BlockSpec(memory_space=pltpu.VMEM),
            pl.BlockSpec(memory_space=pltpu.VMEM),
        ],
        out_specs=pl.BlockSpec(memory_space=pltpu.VMEM),
        out_shape=jax.ShapeDtypeStruct((_B, _D, _N), jnp.float32),
        scratch_shapes=[
            pltpu.VMEM((_B, _C_IN, _N), jnp.float32),
            pltpu.SemaphoreType.DMA,
        ],
    )(z, W, emb)


# speedup vs baseline: 1.0244x; 1.0244x over previous
import jax
import jax.numpy as jnp
from jax.experimental import pallas as pl
from jax.experimental.pallas import tpu as pltpu

_B, _C_IN, _N = 8, 256, 196
_D, _K = 64, 1024
_BN = _B * _N


def _split3(x):
    h = x.astype(jnp.bfloat16)
    r = x - h.astype(jnp.float32)
    m = r.astype(jnp.bfloat16)
    l = (r - m.astype(jnp.float32)).astype(jnp.bfloat16)
    return h, m, l


def _vq_body(z_hbm, w_ref, emb_ref, out_ref, zbuf, sem):
    zcopy = pltpu.make_async_copy(z_hbm, zbuf, sem)
    zcopy.start()

    emb = emb_ref[...]
    eh = emb.astype(jnp.bfloat16)
    el = (emb - eh.astype(jnp.float32)).astype(jnp.bfloat16)
    e_sq = jnp.sum(emb * emb, axis=1, keepdims=True)
    qh, qm, ql = _split3(e_sq)
    lhs = jnp.concatenate([-2.0 * eh, -2.0 * eh, -2.0 * el, qh, qm, ql],
                          axis=1)
    gather_lhs = jnp.concatenate([eh.T, el.T], axis=0)

    wb = w_ref[...].astype(jnp.bfloat16)
    zcopy.wait()
    z_all = jnp.concatenate([zbuf[b] for b in range(_B)], axis=1)
    ze = jnp.dot(wb, z_all.astype(jnp.bfloat16),
                 preferred_element_type=jnp.float32)

    zh = ze.astype(jnp.bfloat16)
    zl = (ze - zh.astype(jnp.float32)).astype(jnp.bfloat16)
    ones = jnp.ones((1, _BN), dtype=jnp.bfloat16)
    rhs = jnp.concatenate([zh, zl, zh, ones, ones, ones], axis=0)
    s = jnp.dot(lhs, rhs, preferred_element_type=jnp.float32)

    m = jnp.min(s, axis=0, keepdims=True)
    kio = jax.lax.broadcasted_iota(jnp.int32, (_K, _BN), 0)
    idx = jnp.min(jnp.where(s <= m, kio, _K), axis=0)
    onehot = (kio == idx[None, :]).astype(jnp.bfloat16)
    zq2 = jnp.dot(gather_lhs, onehot,
                  preferred_element_type=jnp.float32)
    zq = zq2[:_D] + zq2[_D:]
    for b in range(_B):
        out_ref[b] = zq[:, b * _N:(b + 1) * _N]


def kernel(z, W, emb):
    return pl.pallas_call(
        _vq_body,
        in_specs=[
            pl.BlockSpec(memory_space=pl.ANY),
            pl.BlockSpec(memory_space=pltpu.VMEM),
            pl.BlockSpec(memory_space=pltpu.VMEM),
        ],
        out_specs=pl.BlockSpec(memory_space=pltpu.VMEM),
        out_shape=jax.ShapeDtypeStruct((_B, _D, _N), jnp.float32),
        scratch_shapes=[
            pltpu.VMEM((_B, _C_IN, _N), jnp.float32),
            pltpu.SemaphoreType.DMA,
        ],
    )(z, W, emb)


# speedup vs baseline: 1.1627x; 1.1349x over previous
import jax
import jax.numpy as jnp
from jax.experimental import pallas as pl
from jax.experimental.pallas import tpu as pltpu

_B, _C_IN, _N = 8, 256, 196
_D, _K = 64, 1024
_BN = _B * _N


def _split3(x):
    h = x.astype(jnp.bfloat16)
    r = x - h.astype(jnp.float32)
    m = r.astype(jnp.bfloat16)
    l = (r - m.astype(jnp.float32)).astype(jnp.bfloat16)
    return h, m, l


def _vq_body(z_ref, w_ref, emb_ref, out_ref):
    emb = emb_ref[...]

    wb = w_ref[...].astype(jnp.bfloat16)
    z_all = jnp.concatenate([z_ref[b] for b in range(_B)], axis=1)
    ze = jnp.dot(wb, z_all.astype(jnp.bfloat16),
                 preferred_element_type=jnp.float32)

    eh = emb.astype(jnp.bfloat16)
    el = (emb - eh.astype(jnp.float32)).astype(jnp.bfloat16)
    e_sq = jnp.sum(emb * emb, axis=1, keepdims=True)
    qh, qm, ql = _split3(e_sq)
    zh = ze.astype(jnp.bfloat16)
    zl = (ze - zh.astype(jnp.float32)).astype(jnp.bfloat16)

    lhs = jnp.concatenate([-2.0 * eh, -2.0 * eh, -2.0 * el, qh, qm, ql],
                          axis=1)
    ones = jnp.ones((1, _BN), dtype=jnp.bfloat16)
    rhs = jnp.concatenate([zh, zl, zh, ones, ones, ones], axis=0)
    s = jnp.dot(lhs, rhs, preferred_element_type=jnp.float32)

    kio = jax.lax.broadcasted_iota(jnp.int32, (_K, _BN), 0)
    idx = jnp.argmin(s, axis=0).astype(jnp.int32)
    onehot = (kio == idx[None, :]).astype(jnp.bfloat16)
    zq2 = jnp.dot(jnp.concatenate([eh.T, el.T], axis=0), onehot,
                  preferred_element_type=jnp.float32)
    zq = zq2[:_D] + zq2[_D:]
    for b in range(_B):
        out_ref[b] = zq[:, b * _N:(b + 1) * _N]


def kernel(z, W, emb):
    return pl.pallas_call(
        _vq_body,
        in_specs=[
            pl.BlockSpec(memory_space=pltpu.VMEM),
            pl.BlockSpec(memory_space=pltpu.VMEM),
            pl.BlockSpec(memory_space=pltpu.VMEM),
        ],
        out_specs=pl.BlockSpec(memory_space=pltpu.VMEM),
        out_shape=jax.ShapeDtypeStruct((_B, _D, _N), jnp.float32),
    )(z, W, emb)
